# Initial kernel scaffold; baseline (speedup 1.0000x reference)
import jax
import jax.numpy as jnp
from jax import lax
from jax.experimental import pallas as pl
from jax.experimental.pallas import tpu as pltpu
from jax.experimental.pallas import tpu_sc as plsc

N = 10000
E = 320000
D = 128
H = 2
C = 64
ED = 32
SCALE = 0.125

NC = 2
NS = 16
NT = NC * NS
EPT = E // NT
B = 80
NB = EPT // B
RPT = N // NS
RZ = 25

BN = 2000
f32 = jnp.float32


def _proj_body(x_ref, wq_ref, bq_ref, wk_ref, bk_ref, wv_ref, bv_ref, we_ref,
               ws_ref, bs_ref, q_ref, k_ref, v_ref, g_ref, s_ref):
  xb = x_ref[...]
  q = xb @ wq_ref[...] + bq_ref[...]
  q_ref[...] = q
  k_ref[...] = xb @ wk_ref[...] + bk_ref[...]
  v_ref[...] = xb @ wv_ref[...] + bv_ref[...]
  s_ref[...] = xb @ ws_ref[...] + bs_ref[...]
  we = we_ref[...]
  g0 = lax.dot_general(q[:, 0:C], we[:, 0:C], (((1,), (1,)), ((), ())))
  g1 = lax.dot_general(q[:, C:2 * C], we[:, C:2 * C], (((1,), (1,)), ((), ())))
  g_ref[...] = jnp.concatenate([g0, g1], axis=1)


def _project(x, Wq, bq, Wk, bk, Wv, bv, We, Wskip, bskip):
  full = lambda shape: pl.BlockSpec(shape, lambda i: (0, 0))
  row = lambda w: pl.BlockSpec((BN, w), lambda i: (i, 0))
  return pl.pallas_call(
      _proj_body,
      grid=(N // BN,),
      in_specs=[
          row(D), full((D, D)), full((1, D)), full((D, D)), full((1, D)),
          full((D, D)), full((1, D)), full((ED, D)), full((D, D)),
          full((1, D)),
      ],
      out_specs=[row(D), row(D), row(D), row(2 * ED), row(D)],
      out_shape=[
          jax.ShapeDtypeStruct((N, D), f32),
          jax.ShapeDtypeStruct((N, D), f32),
          jax.ShapeDtypeStruct((N, D), f32),
          jax.ShapeDtypeStruct((N, 2 * ED), f32),
          jax.ShapeDtypeStruct((N, D), f32),
      ],
  )(x, Wq, bq.reshape(1, D), Wk, bk.reshape(1, D), Wv, bv.reshape(1, D), We,
    Wskip, bskip.reshape(1, D))


def _edge_body(qt, kt, vt, gt, eat, srct, dstt, outv_hbm, acc_hbm, den_hbm,
               qrows, krows, vrows, grows, earows, srcv, dstv, ostage, astage,
               a0b, a1b, denf, zb, zb2, outsp, accsp, sq, sk, sv, sg):
  c = lax.axis_index("c")
  s = lax.axis_index("s")
  wid = c * NS + s
  zero = jnp.zeros((16,), f32)

  @pl.loop(0, RZ)
  def _fill_zb(r):
    for ch in range(D // 16):
      zb[r, pl.ds(ch * 16, 16)] = zero
    for ch in range(2 * ED // 16):
      zb2[r, pl.ds(ch * 16, 16)] = zero

  row0 = s * RPT

  @pl.loop(0, RPT // RZ)
  def _zero_spmem(r):
    pltpu.sync_copy(zb, outsp.at[pl.ds(row0 + r * RZ, RZ)])
    pltpu.sync_copy(zb2, accsp.at[pl.ds(row0 + r * RZ, RZ)])

  @pl.loop(0, 2 * N // 16)
  def _zero_den(i):
    denf[pl.ds(i * 16, 16)] = zero

  plsc.subcore_barrier()

  base = wid * EPT

  @pl.loop(0, NB)
  def _block(b):
    off = base + b * B
    pltpu.sync_copy(srct.at[pl.ds(off, B)], srcv)
    pltpu.sync_copy(dstt.at[pl.ds(off, B)], dstv)
    cq = pltpu.async_copy(qt.at[dstv], qrows, sq)
    ck = pltpu.async_copy(kt.at[srcv], krows, sk)
    cv = pltpu.async_copy(vt.at[srcv], vrows, sv)
    cg = pltpu.async_copy(gt.at[dstv], grows, sg)
    pltpu.sync_copy(eat.at[pl.ds(off, B)], earows)
    cq.wait()
    ck.wait()
    cv.wait()
    cg.wait()

    @pl.loop(0, B // 16)
    def _group(g):
      j0 = g * 16
      for jj in range(16):
        j = j0 + jj
        acc0 = qrows[j, pl.ds(0, 16)] * krows[j, pl.ds(0, 16)]
        for ch in range(1, 4):
          acc0 += qrows[j, pl.ds(ch * 16, 16)] * krows[j, pl.ds(ch * 16, 16)]
        acc1 = qrows[j, pl.ds(64, 16)] * krows[j, pl.ds(64, 16)]
        for ch in range(5, 8):
          acc1 += qrows[j, pl.ds(ch * 16, 16)] * krows[j, pl.ds(ch * 16, 16)]
        ea0 = earows[j, pl.ds(0, 16)]
        ea1 = earows[j, pl.ds(16, 16)]
        acc0 += ea0 * grows[j, pl.ds(0, 16)]
        acc0 += ea1 * grows[j, pl.ds(16, 16)]
        acc1 += ea0 * grows[j, pl.ds(32, 16)]
        acc1 += ea1 * grows[j, pl.ds(48, 16)]
        a0b[j] = jnp.sum(acc0)
        a1b[j] = jnp.sum(acc1)
      idx16 = dstv[pl.ds(j0, 16)]
      p0 = jnp.exp(a0b[pl.ds(j0, 16)] * SCALE)
      p1 = jnp.exp(a1b[pl.ds(j0, 16)] * SCALE)
      a0b[pl.ds(j0, 16)] = p0
      a1b[pl.ds(j0, 16)] = p1
      idx2 = idx16 * 2
      plsc.addupdate_scatter(denf, [idx2], p0)
      plsc.addupdate_scatter(denf, [idx2 + 1], p1)
      for jj in range(16):
        j = j0 + jj
        p0s = a0b[j]
        p1s = a1b[j]
        for ch in range(4):
          ostage[j, pl.ds(ch * 16, 16)] = p0s * vrows[j, pl.ds(ch * 16, 16)]
        for ch in range(4, 8):
          ostage[j, pl.ds(ch * 16, 16)] = p1s * vrows[j, pl.ds(ch * 16, 16)]
        astage[j, pl.ds(0, 16)] = p0s * earows[j, pl.ds(0, 16)]
        astage[j, pl.ds(16, 16)] = p0s * earows[j, pl.ds(16, 16)]
        astage[j, pl.ds(32, 16)] = p1s * earows[j, pl.ds(0, 16)]
        astage[j, pl.ds(48, 16)] = p1s * earows[j, pl.ds(16, 16)]

    pltpu.sync_copy(ostage, outsp.at[dstv], add=True)
    pltpu.sync_copy(astage, accsp.at[dstv], add=True)

  plsc.subcore_barrier()

  pltpu.sync_copy(outsp.at[pl.ds(row0, RPT)], outv_hbm.at[c, pl.ds(row0, RPT)])
  pltpu.sync_copy(accsp.at[pl.ds(row0, RPT)], acc_hbm.at[c, pl.ds(row0, RPT)])
  pltpu.sync_copy(denf, den_hbm.at[wid])


def _edge_pass(q_tab, k_tab, v_tab, g_tab, ea, src, dst):
  kfn = pl.kernel(
      _edge_body,
      out_type=(
          jax.ShapeDtypeStruct((NC, N, D), f32),
          jax.ShapeDtypeStruct((NC, N, 2 * ED), f32),
          jax.ShapeDtypeStruct((NT, 2 * N), f32),
      ),
      mesh=plsc.VectorSubcoreMesh(core_axis_name="c", subcore_axis_name="s"),
      scratch_types=[
          pltpu.VMEM((B, D), f32),
          pltpu.VMEM((B, D), f32),
          pltpu.VMEM((B, D), f32),
          pltpu.VMEM((B, 2 * ED), f32),
          pltpu.VMEM((B, ED), f32),
          pltpu.VMEM((B,), jnp.int32),
          pltpu.VMEM((B,), jnp.int32),
          pltpu.VMEM((B, D), f32),
          pltpu.VMEM((B, 2 * ED), f32),
          pltpu.VMEM((B,), f32),
          pltpu.VMEM((B,), f32),
          pltpu.VMEM((2 * N,), f32),
          pltpu.VMEM((RZ, D), f32),
          pltpu.VMEM((RZ, 2 * ED), f32),
          pltpu.VMEM_SHARED((N, D), f32),
          pltpu.VMEM_SHARED((N, 2 * ED), f32),
          pltpu.SemaphoreType.DMA,
          pltpu.SemaphoreType.DMA,
          pltpu.SemaphoreType.DMA,
          pltpu.SemaphoreType.DMA,
      ],
  )
  return kfn(q_tab, k_tab, v_tab, g_tab, ea, src, dst)


def _comb_body(ovp_ref, acp_ref, denp_ref, skip_ref, we_ref, out_ref):
  ov = ovp_ref[0] + ovp_ref[1]
  ac = acp_ref[0] + acp_ref[1]
  den = jnp.sum(denp_ref[...], axis=0)
  we = we_ref[...]
  e0 = lax.dot_general(ac[:, 0:ED], we[:, 0:C], (((1,), (0,)), ((), ())))
  e1 = lax.dot_general(ac[:, ED:2 * ED], we[:, C:2 * C],
                       (((1,), (0,)), ((), ())))
  d0 = den[:, 0:1] + 1e-16
  d1 = den[:, 1:2] + 1e-16
  o0 = (ov[:, 0:C] + e0) / d0
  o1 = (ov[:, C:2 * C] + e1) / d1
  out_ref[...] = jnp.concatenate([o0, o1], axis=1) + skip_ref[...]


def _combine(ovp, acp, denp, skip, We):
  return pl.pallas_call(
      _comb_body,
      grid=(N // BN,),
      in_specs=[
          pl.BlockSpec((NC, BN, D), lambda i: (0, i, 0)),
          pl.BlockSpec((NC, BN, 2 * ED), lambda i: (0, i, 0)),
          pl.BlockSpec((NT, BN, 2), lambda i: (0, i, 0)),
          pl.BlockSpec((BN, D), lambda i: (i, 0)),
          pl.BlockSpec((ED, D), lambda i: (0, 0)),
      ],
      out_specs=pl.BlockSpec((BN, D), lambda i: (i, 0)),
      out_shape=jax.ShapeDtypeStruct((N, D), f32),
  )(ovp, acp, denp.reshape(NT, N, 2), skip, We)


def kernel(x, last_update, edge_index, t, msg, Wq, bq, Wk, bk, Wv, bv, We,
           Wskip, bskip):
  del last_update
  ea = jnp.concatenate([t, msg], axis=-1)
  src = edge_index[0]
  dst = edge_index[1]
  q_tab, k_tab, v_tab, g_tab, skip = _project(x, Wq, bq, Wk, bk, Wv, bv, We,
                                              Wskip, bskip)
  ovp, acp, denp = _edge_pass(q_tab, k_tab, v_tab, g_tab, ea, src, dst)
  return _combine(ovp, acp, denp, skip, We)


# speedup vs baseline: 29.6591x; 29.6591x over previous
import jax
import jax.numpy as jnp
from jax import lax
from jax.experimental import pallas as pl
from jax.experimental.pallas import tpu as pltpu
from jax.experimental.pallas import tpu_sc as plsc

N = 10000
E = 320000
D = 128
H = 2
C = 64
ED = 32
SCALE = 0.125

NC = 2
NS = 16
EPT = E // NS
B = 80
NB = EPT // B
RPT = 624
RZ = 80

BN = 2000
f32 = jnp.float32


def _proj_body(x_ref, wq_ref, bq_ref, wk_ref, bk_ref, wv_ref, bv_ref, we_ref,
               ws_ref, bs_ref, qg_ref, kv_ref, s_ref):
  h = pl.program_id(0)
  xb = x_ref[...]
  q = xb @ wq_ref[...] + bq_ref[...]
  k = xb @ wk_ref[...] + bk_ref[...]
  v = xb @ wv_ref[...] + bv_ref[...]
  s_ref[...] = xb @ ws_ref[...] + bs_ref[...]
  we = we_ref[...]
  g0 = lax.dot_general(q[:, 0:C], we[:, 0:C], (((1,), (1,)), ((), ())))
  g1 = lax.dot_general(q[:, C:2 * C], we[:, C:2 * C], (((1,), (1,)), ((), ())))
  qh = jnp.where(h == 0, q[:, 0:C], q[:, C:2 * C])
  kh = jnp.where(h == 0, k[:, 0:C], k[:, C:2 * C])
  vh = jnp.where(h == 0, v[:, 0:C], v[:, C:2 * C])
  gh = jnp.where(h == 0, g0, g1)
  qg_ref[...] = jnp.concatenate([qh, gh, jnp.zeros((BN, ED), f32)], axis=1)
  kv_ref[...] = jnp.concatenate([kh, vh], axis=1)


def _project(x, Wq, bq, Wk, bk, Wv, bv, We, Wskip, bskip):
  full = lambda shape: pl.BlockSpec(shape, lambda h, i: (0, 0))
  rowx = pl.BlockSpec((BN, D), lambda h, i: (i, 0))
  rowh = pl.BlockSpec((BN, D), lambda h, i: (h * (N // BN) + i, 0))
  return pl.pallas_call(
      _proj_body,
      grid=(H, N // BN),
      in_specs=[
          rowx, full((D, D)), full((1, D)), full((D, D)), full((1, D)),
          full((D, D)), full((1, D)), full((ED, D)), full((D, D)),
          full((1, D)),
      ],
      out_specs=[rowh, rowh, rowx],
      out_shape=[
          jax.ShapeDtypeStruct((H * N, D), f32),
          jax.ShapeDtypeStruct((H * N, D), f32),
          jax.ShapeDtypeStruct((N, D), f32),
      ],
  )(x, Wq, bq.reshape(1, D), Wk, bk.reshape(1, D), Wv, bv.reshape(1, D), We,
    Wskip, bskip.reshape(1, D))


def _edge_body(qgt, kvt, eat, srct, dstt, out_hbm,
               qgrows, kvrows, earows, srcv, dstv, dstg, stage, zb, accsp,
               sq, sk):
  c = lax.axis_index("c")
  s = lax.axis_index("s")
  cn = c * N
  ii = lax.iota(jnp.int32, 16)
  zero = ii.astype(f32) * 0.0

  @pl.loop(0, RZ)
  def _fill_zb(r):
    for ch in range(D // 16):
      zb[r, pl.ds(ch * 16, 16)] = zero

  @pl.loop(0, B)
  def _fill_stage_pad(j):
    stage[j, pl.ds(7 * 16, 16)] = zero

  row0 = s * RPT

  @pl.loop(0, 8)
  def _zero_spmem(r):
    pltpu.sync_copy(zb, accsp.at[pl.ds(row0 + r * RZ, RZ)])

  plsc.subcore_barrier()

  base = s * EPT

  @pl.loop(0, NB)
  def _block(b):
    off = base + b * B
    pltpu.sync_copy(srct.at[pl.ds(off, B)], srcv)
    pltpu.sync_copy(dstt.at[pl.ds(off, B)], dstv)

    @pl.loop(0, B // 16)
    def _bias(i):
      srcv[pl.ds(i * 16, 16)] = srcv[pl.ds(i * 16, 16)] + cn
      dstg[pl.ds(i * 16, 16)] = dstv[pl.ds(i * 16, 16)] + cn

    cq = pltpu.async_copy(qgt.at[dstg], qgrows, sq)
    ck = pltpu.async_copy(kvt.at[srcv], kvrows, sk)
    pltpu.sync_copy(eat.at[pl.ds(off, B)], earows)
    cq.wait()
    ck.wait()

    @pl.loop(0, B // 16)
    def _group(g):
      j0 = g * 16
      av = zero
      for jj in range(16):
        j = j0 + jj
        acc = qgrows[j, pl.ds(0, 16)] * kvrows[j, pl.ds(0, 16)]
        for ch in range(1, 4):
          acc += qgrows[j, pl.ds(ch * 16, 16)] * kvrows[j, pl.ds(ch * 16, 16)]
        acc += earows[j, pl.ds(0, 16)] * qgrows[j, pl.ds(C, 16)]
        acc += earows[j, pl.ds(16, 16)] * qgrows[j, pl.ds(C + 16, 16)]
        av = jnp.where(ii == jj, jnp.sum(acc), av)
      p = jnp.exp(av * SCALE)
      for jj in range(16):
        j = j0 + jj
        pb = jnp.full((16,), p[jj], f32)
        for ch in range(4):
          stage[j, pl.ds(ch * 16, 16)] = pb * kvrows[j, pl.ds(C + ch * 16, 16)]
        stage[j, pl.ds(64, 16)] = pb * earows[j, pl.ds(0, 16)]
        stage[j, pl.ds(80, 16)] = pb * earows[j, pl.ds(16, 16)]
        stage[j, pl.ds(96, 16)] = jnp.where(ii == 0, pb, 0.0)

    pltpu.sync_copy(stage, accsp.at[dstv], add=True)

  plsc.subcore_barrier()

  @pl.when(s < NS - 1)
  def _flush_body():
    pltpu.sync_copy(accsp.at[pl.ds(row0, RPT)],
                    out_hbm.at[c, pl.ds(row0, RPT)])

  @pl.when(s == NS - 1)
  def _flush_tail():
    pltpu.sync_copy(accsp.at[pl.ds((NS - 1) * RPT, N - (NS - 1) * RPT)],
                    out_hbm.at[c, pl.ds((NS - 1) * RPT, N - (NS - 1) * RPT)])


def _edge_pass(qg_tab, kv_tab, ea, src, dst):
  kfn = pl.kernel(
      _edge_body,
      out_type=jax.ShapeDtypeStruct((NC, N, D), f32),
      mesh=plsc.VectorSubcoreMesh(core_axis_name="c", subcore_axis_name="s"),
      compiler_params=pltpu.CompilerParams(needs_layout_passes=False,
                                           use_tc_tiling_on_sc=False),
      scratch_types=[
          pltpu.VMEM((B, D), f32),
          pltpu.VMEM((B, D), f32),
          pltpu.VMEM((B, ED), f32),
          pltpu.VMEM((B,), jnp.int32),
          pltpu.VMEM((B,), jnp.int32),
          pltpu.VMEM((B,), jnp.int32),
          pltpu.VMEM((B, D), f32),
          pltpu.VMEM((RZ, D), f32),
          pltpu.VMEM_SHARED((N, D), f32),
          pltpu.SemaphoreType.DMA,
          pltpu.SemaphoreType.DMA,
      ],
  )
  return kfn(qg_tab, kv_tab, ea, src, dst)


def _comb_body(ovp_ref, skip_ref, we_ref, out_ref):
  ov0 = ovp_ref[0]
  ov1 = ovp_ref[1]
  we = we_ref[...]
  e0 = lax.dot_general(ov0[:, C:C + ED], we[:, 0:C], (((1,), (0,)), ((), ())))
  e1 = lax.dot_general(ov1[:, C:C + ED], we[:, C:2 * C],
                       (((1,), (0,)), ((), ())))
  o0 = (ov0[:, 0:C] + e0) / (ov0[:, 96:97] + 1e-16)
  o1 = (ov1[:, 0:C] + e1) / (ov1[:, 96:97] + 1e-16)
  out_ref[...] = jnp.concatenate([o0, o1], axis=1) + skip_ref[...]


def _combine(ovp, skip, We):
  return pl.pallas_call(
      _comb_body,
      grid=(N // BN,),
      in_specs=[
          pl.BlockSpec((NC, BN, D), lambda i: (0, i, 0)),
          pl.BlockSpec((BN, D), lambda i: (i, 0)),
          pl.BlockSpec((ED, D), lambda i: (0, 0)),
      ],
      out_specs=pl.BlockSpec((BN, D), lambda i: (i, 0)),
      out_shape=jax.ShapeDtypeStruct((N, D), f32),
  )(ovp, skip, We)


def kernel(x, last_update, edge_index, t, msg, Wq, bq, Wk, bk, Wv, bv, We,
           Wskip, bskip):
  del last_update
  ea = jnp.concatenate([t, msg], axis=-1)
  src = edge_index[0]
  dst = edge_index[1]
  qg_tab, kv_tab, skip = _project(x, Wq, bq, Wk, bk, Wv, bv, We, Wskip, bskip)
  ovp = _edge_pass(qg_tab, kv_tab, ea, src, dst)
  return _combine(ovp, skip, We)


# speedup vs baseline: 43.1003x; 1.4532x over previous
import jax
import jax.numpy as jnp
from jax import lax
from jax.experimental import pallas as pl
from jax.experimental.pallas import tpu as pltpu
from jax.experimental.pallas import tpu_sc as plsc

N = 10000
E = 320000
D = 128
H = 2
C = 64
ED = 32
SCALE = 0.125

NC = 2
NS = 16
EPT = E // NS
B = 32
NB = EPT // B
RPT = 624
RZ = 80

BN = 2000
f32 = jnp.float32


def _proj_body(x_ref, wq_ref, bq_ref, wk_ref, bk_ref, wv_ref, bv_ref, we_ref,
               ws_ref, bs_ref, qg_ref, kv_ref, s_ref):
  h = pl.program_id(0)
  xb = x_ref[...]
  q = xb @ wq_ref[...] + bq_ref[...]
  k = xb @ wk_ref[...] + bk_ref[...]
  v = xb @ wv_ref[...] + bv_ref[...]
  s_ref[...] = xb @ ws_ref[...] + bs_ref[...]
  we = we_ref[...]
  g0 = lax.dot_general(q[:, 0:C], we[:, 0:C], (((1,), (1,)), ((), ())))
  g1 = lax.dot_general(q[:, C:2 * C], we[:, C:2 * C], (((1,), (1,)), ((), ())))
  qh = jnp.where(h == 0, q[:, 0:C], q[:, C:2 * C])
  kh = jnp.where(h == 0, k[:, 0:C], k[:, C:2 * C])
  vh = jnp.where(h == 0, v[:, 0:C], v[:, C:2 * C])
  gh = jnp.where(h == 0, g0, g1)
  qg_ref[...] = jnp.concatenate([qh, gh, jnp.zeros((BN, ED), f32)], axis=1)
  kv_ref[...] = jnp.concatenate([kh, vh], axis=1)


def _project(x, Wq, bq, Wk, bk, Wv, bv, We, Wskip, bskip):
  full = lambda shape: pl.BlockSpec(shape, lambda h, i: (0, 0))
  rowx = pl.BlockSpec((BN, D), lambda h, i: (i, 0))
  rowh = pl.BlockSpec((BN, D), lambda h, i: (h * (N // BN) + i, 0))
  return pl.pallas_call(
      _proj_body,
      grid=(H, N // BN),
      in_specs=[
          rowx, full((D, D)), full((1, D)), full((D, D)), full((1, D)),
          full((D, D)), full((1, D)), full((ED, D)), full((D, D)),
          full((1, D)),
      ],
      out_specs=[rowh, rowh, rowx],
      out_shape=[
          jax.ShapeDtypeStruct((H * N, D), f32),
          jax.ShapeDtypeStruct((H * N, D), f32),
          jax.ShapeDtypeStruct((N, D), f32),
      ],
  )(x, Wq, bq.reshape(1, D), Wk, bk.reshape(1, D), Wv, bv.reshape(1, D), We,
    Wskip, bskip.reshape(1, D))


def _edge_body(qgt, kvt, eat, srct, dstt, out_hbm,
               qgrA, kvrA, earA, stgA, qgrB, kvrB, earB, stgB,
               rsA, rdA, sgA, dgA, dsA, rsB, rdB, sgB, dgB, dsB, abuf, accsp,
               sqA, skA, seA, sqB, skB, seB, ssA, ssB, sxsA, sxdA, sxsB, sxdB):
  c = lax.axis_index("c")
  s = lax.axis_index("s")
  cn = c * N
  ii = lax.iota(jnp.int32, 16)
  zero = ii.astype(f32) * 0.0

  @pl.loop(0, B)
  def _fill_z(j):
    for ch in range(D // 16):
      stgA[j, pl.ds(ch * 16, 16)] = zero
      stgB[j, pl.ds(ch * 16, 16)] = zero

  row0 = s * RPT

  @pl.loop(0, RPT // (2 * B) + 1)
  def _zero_spmem(r):
    pltpu.sync_copy(stgA, accsp.at[pl.ds(row0 + r * B, B)])
    pltpu.sync_copy(stgB, accsp.at[pl.ds(row0 + (RPT // (2 * B) + 1 + r) * B,
                                         B)])

  plsc.subcore_barrier()

  base = s * EPT

  def issue_idx(bb, rs, rd, sxs, sxd):
    off = base + bb * B
    pltpu.async_copy(srct.at[pl.ds(off, B)], rs, sxs)
    pltpu.async_copy(dstt.at[pl.ds(off, B)], rd, sxd)

  def wait_idx(bb, rs, rd, sxs, sxd):
    off = base + bb * B
    pltpu.make_async_copy(srct.at[pl.ds(off, B)], rs, sxs).wait()
    pltpu.make_async_copy(dstt.at[pl.ds(off, B)], rd, sxd).wait()

  def fill_gidx(rs, rd, sg, dg):
    for i in range(B // 16):
      sg[pl.ds(i * 16, 16)] = rs[pl.ds(i * 16, 16)] + cn
      dg[pl.ds(i * 16, 16)] = rd[pl.ds(i * 16, 16)] + cn

  def fill_sidx(dg, dsb):
    for i in range(B // 16):
      dsb[pl.ds(i * 16, 16)] = dg[pl.ds(i * 16, 16)] - cn

  def issue_gathers(bb, sg, dg, qgr, kvr, ear, sq, sk, se):
    off = base + bb * B
    pltpu.async_copy(qgt.at[dg], qgr, sq)
    pltpu.async_copy(kvt.at[sg], kvr, sk)
    pltpu.async_copy(eat.at[pl.ds(off, B)], ear, se)

  def wait_gathers(sg, dg, qgr, kvr, ear, sq, sk, se, off):
    pltpu.make_async_copy(qgt.at[dg], qgr, sq).wait()
    pltpu.make_async_copy(kvt.at[sg], kvr, sk).wait()
    pltpu.make_async_copy(eat.at[pl.ds(off, B)], ear, se).wait()

  col15 = ii * 0 + 15

  def compute_block(qgr, kvr, ear, stg):
    @pl.loop(0, B // 16)
    def _group(g):
      j0 = g * 16
      for jj in range(16):
        j = j0 + jj
        acc = qgr[j, pl.ds(0, 16)] * kvr[j, pl.ds(0, 16)]
        for ch in range(1, 4):
          acc += qgr[j, pl.ds(ch * 16, 16)] * kvr[j, pl.ds(ch * 16, 16)]
        acc += ear[j, pl.ds(0, 16)] * qgr[j, pl.ds(C, 16)]
        acc += ear[j, pl.ds(16, 16)] * qgr[j, pl.ds(C + 16, 16)]
        abuf[jj] = jnp.cumsum(acc)
      al = plsc.load_gather(abuf, [ii, col15])
      p = jnp.exp(al * SCALE)
      for jj in range(16):
        j = j0 + jj
        pb = jnp.full((16,), p[jj], f32)
        for ch in range(4):
          stg[j, pl.ds(ch * 16, 16)] = pb * kvr[j, pl.ds(C + ch * 16, 16)]
        stg[j, pl.ds(64, 16)] = pb * ear[j, pl.ds(0, 16)]
        stg[j, pl.ds(80, 16)] = pb * ear[j, pl.ds(16, 16)]
        stg[j, pl.ds(96, 16)] = jnp.where(ii == 0, pb, 0.0)

  NT2 = (NB - 1) // 2

  pltpu.sync_copy(srct.at[pl.ds(base, B)], rsA)
  pltpu.sync_copy(dstt.at[pl.ds(base, B)], rdA)
  fill_gidx(rsA, rdA, sgA, dgA)
  issue_gathers(0, sgA, dgA, qgrA, kvrA, earA, sqA, skA, seA)
  pltpu.sync_copy(srct.at[pl.ds(base + B, B)], rsB)
  pltpu.sync_copy(dstt.at[pl.ds(base + B, B)], rdB)
  fill_gidx(rsB, rdB, sgB, dgB)
  issue_gathers(1, sgB, dgB, qgrB, kvrB, earB, sqB, skB, seB)
  issue_idx(2, rsA, rdA, sxsA, sxdA)
  issue_idx(3, rsB, rdB, sxsB, sxdB)

  @pl.loop(0, NT2)
  def _t(t):
    bb = t * 2
    wait_gathers(sgA, dgA, qgrA, kvrA, earA, sqA, skA, seA, base + bb * B)

    @pl.when(t > 0)
    def _wsA():
      pltpu.make_async_copy(stgA, accsp.at[dsA], ssA).wait()

    fill_sidx(dgA, dsA)
    wait_idx(bb + 2, rsA, rdA, sxsA, sxdA)
    fill_gidx(rsA, rdA, sgA, dgA)

    @pl.when(t < NT2 - 1)
    def _pxA():
      issue_idx(bb + 4, rsA, rdA, sxsA, sxdA)

    compute_block(qgrA, kvrA, earA, stgA)
    pltpu.async_copy(stgA, accsp.at[dsA], ssA, add=True)
    issue_gathers(bb + 2, sgA, dgA, qgrA, kvrA, earA, sqA, skA, seA)

    wait_gathers(sgB, dgB, qgrB, kvrB, earB, sqB, skB, seB,
                 base + (bb + 1) * B)

    @pl.when(t > 0)
    def _wsB():
      pltpu.make_async_copy(stgB, accsp.at[dsB], ssB).wait()

    fill_sidx(dgB, dsB)

    @pl.when(t < NT2 - 1)
    def _pwB():
      wait_idx(bb + 3, rsB, rdB, sxsB, sxdB)
      fill_gidx(rsB, rdB, sgB, dgB)

    @pl.when(t < NT2 - 2)
    def _pxB():
      issue_idx(bb + 5, rsB, rdB, sxsB, sxdB)

    compute_block(qgrB, kvrB, earB, stgB)
    pltpu.async_copy(stgB, accsp.at[dsB], ssB, add=True)

    @pl.when(t < NT2 - 1)
    def _pgB():
      issue_gathers(bb + 3, sgB, dgB, qgrB, kvrB, earB, sqB, skB, seB)

  wait_gathers(sgA, dgA, qgrA, kvrA, earA, sqA, skA, seA, base + (NB - 1) * B)
  pltpu.make_async_copy(stgA, accsp.at[dsA], ssA).wait()
  fill_sidx(dgA, dsA)
  compute_block(qgrA, kvrA, earA, stgA)
  pltpu.async_copy(stgA, accsp.at[dsA], ssA, add=True)
  pltpu.make_async_copy(stgB, accsp.at[dsB], ssB).wait()
  pltpu.make_async_copy(stgA, accsp.at[dsA], ssA).wait()

  plsc.subcore_barrier()

  @pl.when(s < NS - 1)
  def _flush_body():
    pltpu.sync_copy(accsp.at[pl.ds(row0, RPT)],
                    out_hbm.at[c, pl.ds(row0, RPT)])

  @pl.when(s == NS - 1)
  def _flush_tail():
    pltpu.sync_copy(accsp.at[pl.ds((NS - 1) * RPT, N - (NS - 1) * RPT)],
                    out_hbm.at[c, pl.ds((NS - 1) * RPT, N - (NS - 1) * RPT)])


def _edge_pass(qg_tab, kv_tab, ea, src, dst):
  kfn = pl.kernel(
      _edge_body,
      out_type=jax.ShapeDtypeStruct((NC, N, D), f32),
      mesh=plsc.VectorSubcoreMesh(core_axis_name="c", subcore_axis_name="s"),
      compiler_params=pltpu.CompilerParams(needs_layout_passes=False,
                                           use_tc_tiling_on_sc=False),
      scratch_types=(
          [
              pltpu.VMEM((B, D), f32),
              pltpu.VMEM((B, D), f32),
              pltpu.VMEM((B, ED), f32),
              pltpu.VMEM((B, D), f32),
              pltpu.VMEM((B, D), f32),
              pltpu.VMEM((B, D), f32),
              pltpu.VMEM((B, ED), f32),
              pltpu.VMEM((B, D), f32),
          ] + [pltpu.VMEM((B,), jnp.int32)] * 10
          + [
              pltpu.VMEM((16, 16), f32),
              pltpu.VMEM_SHARED((N, D), f32),
          ] + [pltpu.SemaphoreType.DMA] * 12),
  )
  return kfn(qg_tab, kv_tab, ea, src, dst)


def _comb_body(ovp_ref, skip_ref, we_ref, out_ref):
  ov0 = ovp_ref[0]
  ov1 = ovp_ref[1]
  we = we_ref[...]
  e0 = lax.dot_general(ov0[:, C:C + ED], we[:, 0:C], (((1,), (0,)), ((), ())))
  e1 = lax.dot_general(ov1[:, C:C + ED], we[:, C:2 * C],
                       (((1,), (0,)), ((), ())))
  o0 = (ov0[:, 0:C] + e0) / (ov0[:, 96:97] + 1e-16)
  o1 = (ov1[:, 0:C] + e1) / (ov1[:, 96:97] + 1e-16)
  out_ref[...] = jnp.concatenate([o0, o1], axis=1) + skip_ref[...]


def _combine(ovp, skip, We):
  return pl.pallas_call(
      _comb_body,
      grid=(N // BN,),
      in_specs=[
          pl.BlockSpec((NC, BN, D), lambda i: (0, i, 0)),
          pl.BlockSpec((BN, D), lambda i: (i, 0)),
          pl.BlockSpec((ED, D), lambda i: (0, 0)),
      ],
      out_specs=pl.BlockSpec((BN, D), lambda i: (i, 0)),
      out_shape=jax.ShapeDtypeStruct((N, D), f32),
  )(ovp, skip, We)


def kernel(x, last_update, edge_index, t, msg, Wq, bq, Wk, bk, Wv, bv, We,
           Wskip, bskip):
  del last_update
  ea = jnp.concatenate([t, msg], axis=-1)
  src = edge_index[0]
  dst = edge_index[1]
  qg_tab, kv_tab, skip = _project(x, Wq, bq, Wk, bk, Wv, bv, We, Wskip, bskip)
  ovp = _edge_pass(qg_tab, kv_tab, ea, src, dst)
  return _combine(ovp, skip, We)
